# baseline (speedup 1.0000x reference)
import functools

import jax
import jax.numpy as jnp
from jax import lax
from jax.experimental import pallas as pl
from jax.experimental.pallas import tpu as pltpu
from jax.experimental.pallas import tpu_sc as plsc

EPS = 1e-12

NC = 2
NS = 16
NW = NC * NS

CHUNK = 400


def _sc_gather(table, flat_ids):
    n = flat_ids.shape[0]
    d = table.shape[1]
    b_per_w = n // NW
    assert n % (NW * 2 * CHUNK) == 0

    mesh = plsc.VectorSubcoreMesh(core_axis_name="c", subcore_axis_name="s")

    @functools.partial(
        pl.kernel,
        out_type=jax.ShapeDtypeStruct((n, d), jnp.float32),
        mesh=mesh,
        scratch_types=[
            pltpu.VMEM((b_per_w,), jnp.int32),
            pltpu.VMEM((CHUNK, d), jnp.float32),
            pltpu.VMEM((CHUNK, d), jnp.float32),
            pltpu.SemaphoreType.DMA,
            pltpu.SemaphoreType.DMA,
            pltpu.SemaphoreType.DMA,
            pltpu.SemaphoreType.DMA,
        ],
    )
    def k(table_hbm, idx_hbm, out_hbm, idx_v, rows0, rows1, g0s, g1s, s0s, s1s):
        wid = lax.axis_index("s") * NC + lax.axis_index("c")
        base = wid * b_per_w
        pltpu.sync_copy(idx_hbm.at[pl.ds(base, b_per_w)], idx_v)

        @pl.loop(0, b_per_w, step=2 * CHUNK)
        def _(c):
            g0 = pltpu.async_copy(
                table_hbm.at[idx_v.at[pl.ds(c, CHUNK)]], rows0, g0s)
            g1 = pltpu.async_copy(
                table_hbm.at[idx_v.at[pl.ds(c + CHUNK, CHUNK)]], rows1, g1s)
            g0.wait()
            s0 = pltpu.async_copy(rows0, out_hbm.at[pl.ds(base + c, CHUNK)], s0s)
            g1.wait()
            s1 = pltpu.async_copy(
                rows1, out_hbm.at[pl.ds(base + c + CHUNK, CHUNK)], s1s)
            s0.wait()
            s1.wait()

    return k(table, flat_ids)


def _ln_body(g_ref, tt_ref, pos_ref, tok_ref, gamma_ref, beta_ref, o_ref):
    g = g_ref[...]
    tt = tt_ref[...]
    pos = pos_ref[...]
    d = g.shape[-1]
    inv_d = 1.0 / d
    base = pos + tok_ref[0, :]
    dtok = tok_ref[1, :] - tok_ref[0, :]
    ttf = tt.astype(jnp.float32)[..., None]
    emb = g + base[None] + ttf * dtok
    s1 = jnp.sum(emb, axis=-1, keepdims=True)
    s2 = jnp.sum(emb * emb, axis=-1, keepdims=True)
    mean = s1 * inv_d
    var = s2 * inv_d - mean * mean
    scale = lax.rsqrt(var + EPS) * gamma_ref[0, :]
    o_ref[...] = (emb - mean) * scale + beta_ref[0, :]


def kernel(input_ids, token_type_ids, W_word, W_pos, W_tok, gamma, beta):
    B, S = input_ids.shape
    D = W_word.shape[1]
    NSLICES = 8
    BSL = B // NSLICES
    M = 32
    nblk = BSL // M

    ids32 = input_ids.astype(jnp.int32)
    tt32 = token_type_ids.astype(jnp.int32)
    pos_s = W_pos[:S]
    gamma2 = gamma.reshape(1, D)
    beta2 = beta.reshape(1, D)

    def ln_piece(k, g_k, tt_k, prev):
        body = _ln_body if prev is None else (
            lambda p_ref, *rest: _ln_body(*rest))
        in_specs = [
            pl.BlockSpec((M, S, D), lambda i: (i, 0, 0)),
            pl.BlockSpec((M, S), lambda i: (i, 0)),
            pl.BlockSpec((S, D), lambda i: (0, 0)),
            pl.BlockSpec((2, D), lambda i: (0, 0)),
            pl.BlockSpec((1, D), lambda i: (0, 0)),
            pl.BlockSpec((1, D), lambda i: (0, 0)),
        ]
        args = [g_k, tt_k, pos_s, W_tok, gamma2, beta2]
        aliases = {}
        if prev is not None:
            in_specs = [pl.BlockSpec(memory_space=pl.ANY)] + in_specs
            args = [prev] + args
            aliases = {0: 0}
        off = k * nblk
        return pl.pallas_call(
            body,
            grid=(nblk,),
            in_specs=in_specs,
            out_specs=pl.BlockSpec((M, S, D), lambda i: (off + i, 0, 0)),
            out_shape=jax.ShapeDtypeStruct((B, S, D), jnp.float32),
            input_output_aliases=aliases,
        )(*args)

    gathered = [
        _sc_gather(W_word, ids32[k * BSL:(k + 1) * BSL].reshape(-1))
        .reshape(BSL, S, D)
        for k in range(NSLICES)
    ]
    out = None
    for k in range(NSLICES):
        out = ln_piece(k, gathered[k], tt32[k * BSL:(k + 1) * BSL], out)
    return out


# speedup vs baseline: 1.0819x; 1.0819x over previous
import functools

import jax
import jax.numpy as jnp
from jax import lax
from jax.experimental import pallas as pl
from jax.experimental.pallas import tpu as pltpu
from jax.experimental.pallas import tpu_sc as plsc

EPS = 1e-12

NC = 2
NS = 16
NW = NC * NS

CHUNK = 400


def _sc_gather(table, flat_ids):
    n = flat_ids.shape[0]
    d = table.shape[1]
    b_per_w = n // NW
    assert n % (NW * 2 * CHUNK) == 0

    mesh = plsc.VectorSubcoreMesh(core_axis_name="c", subcore_axis_name="s")

    @functools.partial(
        pl.kernel,
        out_type=jax.ShapeDtypeStruct((n, d), jnp.float32),
        mesh=mesh,
        scratch_types=[
            pltpu.VMEM((b_per_w,), jnp.int32),
            pltpu.VMEM((CHUNK, d), jnp.float32),
            pltpu.VMEM((CHUNK, d), jnp.float32),
            pltpu.SemaphoreType.DMA,
            pltpu.SemaphoreType.DMA,
            pltpu.SemaphoreType.DMA,
            pltpu.SemaphoreType.DMA,
        ],
    )
    def k(table_hbm, idx_hbm, out_hbm, idx_v, rows0, rows1, g0s, g1s, s0s, s1s):
        wid = lax.axis_index("s") * NC + lax.axis_index("c")
        base = wid * b_per_w
        pltpu.sync_copy(idx_hbm.at[pl.ds(base, b_per_w)], idx_v)

        @pl.loop(0, b_per_w, step=2 * CHUNK)
        def _(c):
            g0 = pltpu.async_copy(
                table_hbm.at[idx_v.at[pl.ds(c, CHUNK)]], rows0, g0s)
            g1 = pltpu.async_copy(
                table_hbm.at[idx_v.at[pl.ds(c + CHUNK, CHUNK)]], rows1, g1s)
            g0.wait()
            s0 = pltpu.async_copy(rows0, out_hbm.at[pl.ds(base + c, CHUNK)], s0s)
            g1.wait()
            s1 = pltpu.async_copy(
                rows1, out_hbm.at[pl.ds(base + c + CHUNK, CHUNK)], s1s)
            s0.wait()
            s1.wait()

    return k(table, flat_ids)


def _ln_body(g_ref, tt_ref, pos_ref, tok_ref, gamma_ref, beta_ref, o_ref):
    g = g_ref[...]
    tt = tt_ref[...]
    pos = pos_ref[...]
    d = g.shape[-1]
    inv_d = 1.0 / d
    base = pos + tok_ref[0, :]
    dtok = tok_ref[1, :] - tok_ref[0, :]
    ttf = tt.astype(jnp.float32)[..., None]
    emb = g + base[None] + ttf * dtok
    s1 = jnp.sum(emb, axis=-1, keepdims=True)
    s2 = jnp.sum(emb * emb, axis=-1, keepdims=True)
    mean = s1 * inv_d
    var = s2 * inv_d - mean * mean
    scale = lax.rsqrt(var + EPS) * gamma_ref[0, :]
    o_ref[...] = (emb - mean) * scale + beta_ref[0, :]


def kernel(input_ids, token_type_ids, W_word, W_pos, W_tok, gamma, beta):
    B, S = input_ids.shape
    D = W_word.shape[1]
    NSLICES = 4
    BSL = B // NSLICES
    M = 32
    nblk = BSL // M

    ids32 = input_ids.astype(jnp.int32)
    tt32 = token_type_ids.astype(jnp.int32)
    pos_s = W_pos[:S]
    gamma2 = gamma.reshape(1, D)
    beta2 = beta.reshape(1, D)

    def ln_piece(k, g_k, tt_k, prev):
        body = _ln_body if prev is None else (
            lambda p_ref, *rest: _ln_body(*rest))
        in_specs = [
            pl.BlockSpec((M, S, D), lambda i: (i, 0, 0)),
            pl.BlockSpec((M, S), lambda i: (i, 0)),
            pl.BlockSpec((S, D), lambda i: (0, 0)),
            pl.BlockSpec((2, D), lambda i: (0, 0)),
            pl.BlockSpec((1, D), lambda i: (0, 0)),
            pl.BlockSpec((1, D), lambda i: (0, 0)),
        ]
        args = [g_k, tt_k, pos_s, W_tok, gamma2, beta2]
        aliases = {}
        if prev is not None:
            in_specs = [pl.BlockSpec(memory_space=pl.ANY)] + in_specs
            args = [prev] + args
            aliases = {0: 0}
        off = k * nblk
        return pl.pallas_call(
            body,
            grid=(nblk,),
            in_specs=in_specs,
            out_specs=pl.BlockSpec((M, S, D), lambda i: (off + i, 0, 0)),
            out_shape=jax.ShapeDtypeStruct((B, S, D), jnp.float32),
            input_output_aliases=aliases,
        )(*args)

    gathered = [
        _sc_gather(W_word, ids32[k * BSL:(k + 1) * BSL].reshape(-1))
        .reshape(BSL, S, D)
        for k in range(NSLICES)
    ]
    out = None
    for k in range(NSLICES):
        out = ln_piece(k, gathered[k], tt32[k * BSL:(k + 1) * BSL], out)
    return out


# speedup vs baseline: 1.0861x; 1.0039x over previous
import functools

import jax
import jax.numpy as jnp
from jax import lax
from jax.experimental import pallas as pl
from jax.experimental.pallas import tpu as pltpu
from jax.experimental.pallas import tpu_sc as plsc

EPS = 1e-12

NC = 2
NS = 16
NW = NC * NS

CHUNK = 400


def _sc_gather(table, flat_ids):
    n = flat_ids.shape[0]
    d = table.shape[1]
    b_per_w = n // NW
    assert n % (NW * 2 * CHUNK) == 0

    mesh = plsc.VectorSubcoreMesh(core_axis_name="c", subcore_axis_name="s")

    @functools.partial(
        pl.kernel,
        out_type=jax.ShapeDtypeStruct((n, d), jnp.float32),
        mesh=mesh,
        scratch_types=[
            pltpu.VMEM((b_per_w,), jnp.int32),
            pltpu.VMEM((CHUNK, d), jnp.float32),
            pltpu.VMEM((CHUNK, d), jnp.float32),
            pltpu.SemaphoreType.DMA,
            pltpu.SemaphoreType.DMA,
            pltpu.SemaphoreType.DMA,
            pltpu.SemaphoreType.DMA,
        ],
    )
    def k(table_hbm, idx_hbm, out_hbm, idx_v, rows0, rows1, g0s, g1s, s0s, s1s):
        wid = lax.axis_index("s") * NC + lax.axis_index("c")
        base = wid * b_per_w
        pltpu.sync_copy(idx_hbm.at[pl.ds(base, b_per_w)], idx_v)

        @pl.loop(0, b_per_w, step=2 * CHUNK)
        def _(c):
            g0 = pltpu.async_copy(
                table_hbm.at[idx_v.at[pl.ds(c, CHUNK)]], rows0, g0s)
            g1 = pltpu.async_copy(
                table_hbm.at[idx_v.at[pl.ds(c + CHUNK, CHUNK)]], rows1, g1s)
            g0.wait()
            s0 = pltpu.async_copy(rows0, out_hbm.at[pl.ds(base + c, CHUNK)], s0s)
            g1.wait()
            s1 = pltpu.async_copy(
                rows1, out_hbm.at[pl.ds(base + c + CHUNK, CHUNK)], s1s)
            s0.wait()
            s1.wait()

    return k(table, flat_ids)


def _ln_body(g_ref, tt_ref, pos_ref, tok_ref, gamma_ref, beta_ref, o_ref):
    g = g_ref[...]
    tt = tt_ref[...]
    pos = pos_ref[...]
    d = g.shape[-1]
    inv_d = 1.0 / d
    base = pos + tok_ref[0, :]
    dtok = tok_ref[1, :] - tok_ref[0, :]
    ttf = tt.astype(jnp.float32)[..., None]
    emb = g + base[None] + ttf * dtok
    s1 = jnp.sum(emb, axis=-1)
    s2 = jnp.sum(emb * emb, axis=-1)
    mean = s1 * inv_d
    var = s2 * inv_d - mean * mean
    rstd = lax.rsqrt(var + EPS)
    scale = rstd[..., None] * gamma_ref[0, :]
    o_ref[...] = (emb - mean[..., None]) * scale + beta_ref[0, :]


def kernel(input_ids, token_type_ids, W_word, W_pos, W_tok, gamma, beta):
    B, S = input_ids.shape
    D = W_word.shape[1]
    NSLICES = 4
    BSL = B // NSLICES
    M = 32
    nblk = BSL // M

    ids32 = input_ids.astype(jnp.int32)
    tt32 = token_type_ids.astype(jnp.int32)
    pos_s = W_pos[:S]
    gamma2 = gamma.reshape(1, D)
    beta2 = beta.reshape(1, D)

    def ln_piece(k, g_k, tt_k, prev):
        body = _ln_body if prev is None else (
            lambda p_ref, *rest: _ln_body(*rest))
        in_specs = [
            pl.BlockSpec((M, S, D), lambda i: (i, 0, 0)),
            pl.BlockSpec((M, S), lambda i: (i, 0)),
            pl.BlockSpec((S, D), lambda i: (0, 0)),
            pl.BlockSpec((2, D), lambda i: (0, 0)),
            pl.BlockSpec((1, D), lambda i: (0, 0)),
            pl.BlockSpec((1, D), lambda i: (0, 0)),
        ]
        args = [g_k, tt_k, pos_s, W_tok, gamma2, beta2]
        aliases = {}
        if prev is not None:
            in_specs = [pl.BlockSpec(memory_space=pl.ANY)] + in_specs
            args = [prev] + args
            aliases = {0: 0}
        off = k * nblk
        return pl.pallas_call(
            body,
            grid=(nblk,),
            in_specs=in_specs,
            out_specs=pl.BlockSpec((M, S, D), lambda i: (off + i, 0, 0)),
            out_shape=jax.ShapeDtypeStruct((B, S, D), jnp.float32),
            input_output_aliases=aliases,
        )(*args)

    gathered = [
        _sc_gather(W_word, ids32[k * BSL:(k + 1) * BSL].reshape(-1))
        .reshape(BSL, S, D)
        for k in range(NSLICES)
    ]
    out = None
    for k in range(NSLICES):
        out = ln_piece(k, gathered[k], tt32[k * BSL:(k + 1) * BSL], out)
    return out
